# Initial kernel scaffold; baseline (speedup 1.0000x reference)
import jax
import jax.numpy as jnp
from jax.experimental import pallas as pl


def kernel(node_feat, edge_feat, src_idx, edge_type, emb_update, emb_reset, emb_candidate, W_update, b_update, W_reset, b_reset, W_candidate):
    raise NotImplementedError("write your pallas kernel here")


# speedup vs baseline: 7.0947x; 7.0947x over previous
import functools

import jax
import jax.numpy as jnp
from jax import lax
from jax.experimental import pallas as pl
from jax.experimental.pallas import tpu as pltpu
from jax.experimental.pallas import tpu_sc as plsc

_N = 10000
_E = 320000
_D = 128
_R = 200
_RP = 256

_NC = 2
_NS = 16
_NW = _NC * _NS
_PER_W = _E // _NW
_C = 128
_NFULL = _PER_W // _C
_TAIL = _PER_W - _NFULL * _C


def _sc_gather_body(node_hbm, idx_hbm, out_hbm, idx_v, rows_v, gsem):
    wid = lax.axis_index("s") * _NC + lax.axis_index("c")
    base = pl.multiple_of(wid * _PER_W, 8)
    pltpu.sync_copy(idx_hbm.at[pl.ds(base, _PER_W)], idx_v)

    def chunk(j, carry):
        off = pl.multiple_of(j * _C, 8)
        pltpu.async_copy(
            node_hbm.at[idx_v.at[pl.ds(off, _C)]], rows_v, gsem
        ).wait()
        pltpu.sync_copy(rows_v, out_hbm.at[pl.ds(base + off, _C)])
        return carry

    lax.fori_loop(0, _NFULL, chunk, 0)
    toff = pl.multiple_of(_NFULL * _C, 8)
    pltpu.async_copy(
        node_hbm.at[idx_v.at[pl.ds(toff, _TAIL)]],
        rows_v.at[pl.ds(0, _TAIL)],
        gsem,
    ).wait()
    pltpu.sync_copy(rows_v.at[pl.ds(0, _TAIL)], out_hbm.at[pl.ds(base + toff, _TAIL)])


def _sc_gather(node_feat, src_idx):
    mesh = plsc.VectorSubcoreMesh(core_axis_name="c", subcore_axis_name="s")
    fn = functools.partial(
        pl.kernel,
        mesh=mesh,
        out_type=jax.ShapeDtypeStruct((_E, _D), jnp.float32),
        scratch_types=[
            pltpu.VMEM((_PER_W,), jnp.int32),
            pltpu.VMEM((_C, _D), jnp.float32),
            pltpu.SemaphoreType.DMA,
        ],
    )(_sc_gather_body)
    return fn(node_feat, src_idx)


_B = 2560


def _tc_body(et_ref, src_ref, ef_ref, emb_ref, wur_ref, wc_ref, bu_ref, br_ref,
             out_ref):
    src = src_ref[...]
    ef = ef_ref[...]
    et = et_ref[...]
    oh = (et == lax.broadcasted_iota(jnp.int32, (_B, _RP), 1)).astype(jnp.float32)
    g = jnp.dot(oh, emb_ref[...], preferred_element_type=jnp.float32)
    pur = jnp.dot(src, wur_ref[...], preferred_element_type=jnp.float32)
    u = jax.nn.sigmoid(g[:, :_D] * ef + pur[:, :_D] + bu_ref[...])
    r = jax.nn.sigmoid(g[:, _D:2 * _D] * ef + pur[:, _D:] + br_ref[...])
    c = jnp.tanh(
        g[:, 2 * _D:] * ef
        + jnp.dot(r * src, wc_ref[...], preferred_element_type=jnp.float32)
    )
    out_ref[...] = u * c + (1.0 - u) * src


def _tc_compute(et2, src, edge_feat, emb_all, wur, wc, bu, br):
    grid = _E // _B
    return pl.pallas_call(
        _tc_body,
        grid=(grid,),
        in_specs=[
            pl.BlockSpec((_B, 1), lambda i: (i, 0)),
            pl.BlockSpec((_B, _D), lambda i: (i, 0)),
            pl.BlockSpec((_B, _D), lambda i: (i, 0)),
            pl.BlockSpec((_RP, 3 * _D), lambda i: (0, 0)),
            pl.BlockSpec((_D, 2 * _D), lambda i: (0, 0)),
            pl.BlockSpec((_D, _D), lambda i: (0, 0)),
            pl.BlockSpec((1, _D), lambda i: (0, 0)),
            pl.BlockSpec((1, _D), lambda i: (0, 0)),
        ],
        out_specs=pl.BlockSpec((_B, _D), lambda i: (i, 0)),
        out_shape=jax.ShapeDtypeStruct((_E, _D), jnp.float32),
    )(et2, src, edge_feat, emb_all, wur, wc, bu, br)


def kernel(node_feat, edge_feat, src_idx, edge_type, emb_update, emb_reset,
           emb_candidate, W_update, b_update, W_reset, b_reset, W_candidate):
    src_idx = src_idx.astype(jnp.int32)
    edge_type = edge_type.astype(jnp.int32)
    src = _sc_gather(node_feat, src_idx)
    emb_all = jnp.pad(
        jnp.concatenate([emb_update, emb_reset, emb_candidate], axis=1),
        ((0, _RP - _R), (0, 0)),
    )
    wur = jnp.concatenate([W_update.T, W_reset.T], axis=1)
    return _tc_compute(
        edge_type.reshape(_E, 1),
        src,
        edge_feat,
        emb_all,
        wur,
        W_candidate.T,
        b_update.reshape(1, _D),
        b_reset.reshape(1, _D),
    )


# speedup vs baseline: 7.4015x; 1.0432x over previous
import functools

import jax
import jax.numpy as jnp
from jax import lax
from jax.experimental import pallas as pl
from jax.experimental.pallas import tpu as pltpu
from jax.experimental.pallas import tpu_sc as plsc

_N = 10000
_E = 320000
_D = 128
_R = 200
_RP = 256

_NC = 2
_NS = 16
_NW = _NC * _NS
_PER_W = _E // _NW
_C = 128
_NFULL = _PER_W // _C
_TAIL = _PER_W - _NFULL * _C


def _sc_gather_body(node_hbm, idx_hbm, out_hbm, idx_v, rows0, rows1, g0, g1):
    wid = lax.axis_index("s") * _NC + lax.axis_index("c")
    base = pl.multiple_of(wid * _PER_W, 8)
    pltpu.sync_copy(idx_hbm.at[pl.ds(base, _PER_W)], idx_v)

    bufs = (rows0, rows1)
    sems = (g0, g1)

    def start(j, b):
        off = pl.multiple_of(j * _C, 8)
        return pltpu.async_copy(
            node_hbm.at[idx_v.at[pl.ds(off, _C)]], bufs[b], sems[b]
        )

    def drain(j, b):
        pltpu.make_async_copy(
            node_hbm.at[idx_v.at[pl.ds(0, _C)]], bufs[b], sems[b]
        ).wait()
        off = pl.multiple_of(j * _C, 8)
        pltpu.sync_copy(bufs[b], out_hbm.at[pl.ds(base + off, _C)])

    start(0, 0)

    def pair(m, carry):
        j0 = 2 * m
        start(j0 + 1, 1)
        drain(j0, 0)

        @pl.when(m + 1 < _NFULL // 2)
        def _():
            start(j0 + 2, 0)

        drain(j0 + 1, 1)
        return carry

    lax.fori_loop(0, _NFULL // 2, pair, 0)
    toff = pl.multiple_of(_NFULL * _C, 8)
    pltpu.async_copy(
        node_hbm.at[idx_v.at[pl.ds(toff, _TAIL)]],
        rows0.at[pl.ds(0, _TAIL)],
        g0,
    ).wait()
    pltpu.sync_copy(rows0.at[pl.ds(0, _TAIL)], out_hbm.at[pl.ds(base + toff, _TAIL)])


def _sc_gather(node_feat, src_idx):
    mesh = plsc.VectorSubcoreMesh(core_axis_name="c", subcore_axis_name="s")
    fn = functools.partial(
        pl.kernel,
        mesh=mesh,
        out_type=jax.ShapeDtypeStruct((_E, _D), jnp.float32),
        scratch_types=[
            pltpu.VMEM((_PER_W,), jnp.int32),
            pltpu.VMEM((_C, _D), jnp.float32),
            pltpu.VMEM((_C, _D), jnp.float32),
            pltpu.SemaphoreType.DMA,
            pltpu.SemaphoreType.DMA,
        ],
    )(_sc_gather_body)
    return fn(node_feat, src_idx)


_B = 2560


def _tc_body(et_ref, src_ref, ef_ref, emb_ref, wur_ref, wc_ref, bu_ref, br_ref,
             out_ref):
    src = src_ref[...]
    ef = ef_ref[...]
    et = et_ref[...]
    oh = (et == lax.broadcasted_iota(jnp.int32, (_B, _RP), 1)).astype(jnp.float32)
    g = jnp.dot(oh, emb_ref[...], preferred_element_type=jnp.float32)
    pur = jnp.dot(src, wur_ref[...], preferred_element_type=jnp.float32)
    u = jax.nn.sigmoid(g[:, :_D] * ef + pur[:, :_D] + bu_ref[...])
    r = jax.nn.sigmoid(g[:, _D:2 * _D] * ef + pur[:, _D:] + br_ref[...])
    c = jnp.tanh(
        g[:, 2 * _D:] * ef
        + jnp.dot(r * src, wc_ref[...], preferred_element_type=jnp.float32)
    )
    out_ref[...] = u * c + (1.0 - u) * src


def _tc_compute(et2, src, edge_feat, emb_all, wur, wc, bu, br):
    grid = _E // _B
    return pl.pallas_call(
        _tc_body,
        grid=(grid,),
        in_specs=[
            pl.BlockSpec((_B, 1), lambda i: (i, 0)),
            pl.BlockSpec((_B, _D), lambda i: (i, 0)),
            pl.BlockSpec((_B, _D), lambda i: (i, 0)),
            pl.BlockSpec((_RP, 3 * _D), lambda i: (0, 0)),
            pl.BlockSpec((_D, 2 * _D), lambda i: (0, 0)),
            pl.BlockSpec((_D, _D), lambda i: (0, 0)),
            pl.BlockSpec((1, _D), lambda i: (0, 0)),
            pl.BlockSpec((1, _D), lambda i: (0, 0)),
        ],
        out_specs=pl.BlockSpec((_B, _D), lambda i: (i, 0)),
        out_shape=jax.ShapeDtypeStruct((_E, _D), jnp.float32),
    )(et2, src, edge_feat, emb_all, wur, wc, bu, br)


def kernel(node_feat, edge_feat, src_idx, edge_type, emb_update, emb_reset,
           emb_candidate, W_update, b_update, W_reset, b_reset, W_candidate):
    src_idx = src_idx.astype(jnp.int32)
    edge_type = edge_type.astype(jnp.int32)
    src = _sc_gather(node_feat, src_idx)
    emb_all = jnp.pad(
        jnp.concatenate([emb_update, emb_reset, emb_candidate], axis=1),
        ((0, _RP - _R), (0, 0)),
    )
    wur = jnp.concatenate([W_update.T, W_reset.T], axis=1)
    return _tc_compute(
        edge_type.reshape(_E, 1),
        src,
        edge_feat,
        emb_all,
        wur,
        W_candidate.T,
        b_update.reshape(1, _D),
        b_reset.reshape(1, _D),
    )


# speedup vs baseline: 8.1582x; 1.1022x over previous
import functools

import jax
import jax.numpy as jnp
from jax import lax
from jax.experimental import pallas as pl
from jax.experimental.pallas import tpu as pltpu
from jax.experimental.pallas import tpu_sc as plsc

_N = 10000
_E = 320000
_D = 128
_R = 200
_RP = 256

_NC = 2
_NS = 16
_NW = _NC * _NS
_PER_W = _E // _NW
_C = 128
_NFULL = _PER_W // _C
_TAIL = _PER_W - _NFULL * _C


_STAGE = 632
_STAGE_LAST = _N - 15 * _STAGE


def _sc_gather_body(node_hbm, idx_hbm, out_hbm, table_sp, idx_v, rows0, rows1,
                    g0, g1):
    sid = lax.axis_index("s")
    wid = sid * _NC + lax.axis_index("c")
    base = pl.multiple_of(wid * _PER_W, 8)
    @pl.when(sid < _NS - 1)
    def _():
        soff = pl.multiple_of(sid * _STAGE, 8)
        pltpu.sync_copy(
            node_hbm.at[pl.ds(soff, _STAGE)],
            table_sp.at[pl.ds(soff, _STAGE)],
        )

    @pl.when(sid == _NS - 1)
    def _():
        pltpu.sync_copy(
            node_hbm.at[pl.ds(15 * _STAGE, _STAGE_LAST)],
            table_sp.at[pl.ds(15 * _STAGE, _STAGE_LAST)],
        )
    pltpu.sync_copy(idx_hbm.at[pl.ds(base, _PER_W)], idx_v)
    plsc.subcore_barrier()

    bufs = (rows0, rows1)
    sems = (g0, g1)

    def start(j, b):
        off = pl.multiple_of(j * _C, 8)
        return pltpu.async_copy(
            table_sp.at[idx_v.at[pl.ds(off, _C)]], bufs[b], sems[b]
        )

    def drain(j, b):
        pltpu.make_async_copy(
            node_hbm.at[idx_v.at[pl.ds(0, _C)]], bufs[b], sems[b]
        ).wait()
        off = pl.multiple_of(j * _C, 8)
        pltpu.sync_copy(bufs[b], out_hbm.at[pl.ds(base + off, _C)])

    start(0, 0)

    def pair(m, carry):
        j0 = 2 * m
        start(j0 + 1, 1)
        drain(j0, 0)

        @pl.when(m + 1 < _NFULL // 2)
        def _():
            start(j0 + 2, 0)

        drain(j0 + 1, 1)
        return carry

    lax.fori_loop(0, _NFULL // 2, pair, 0)
    toff = pl.multiple_of(_NFULL * _C, 8)
    pltpu.async_copy(
        table_sp.at[idx_v.at[pl.ds(toff, _TAIL)]],
        rows0.at[pl.ds(0, _TAIL)],
        g0,
    ).wait()
    pltpu.sync_copy(rows0.at[pl.ds(0, _TAIL)], out_hbm.at[pl.ds(base + toff, _TAIL)])


def _sc_gather(node_feat, src_idx):
    mesh = plsc.VectorSubcoreMesh(core_axis_name="c", subcore_axis_name="s")
    fn = functools.partial(
        pl.kernel,
        mesh=mesh,
        out_type=jax.ShapeDtypeStruct((_E, _D), jnp.float32),
        scratch_types=[
            pltpu.VMEM_SHARED((_N, _D), jnp.float32),
            pltpu.VMEM((_PER_W,), jnp.int32),
            pltpu.VMEM((_C, _D), jnp.float32),
            pltpu.VMEM((_C, _D), jnp.float32),
            pltpu.SemaphoreType.DMA,
            pltpu.SemaphoreType.DMA,
        ],
    )(_sc_gather_body)
    return fn(node_feat, src_idx)


_B = 2560


def _tc_body(et_ref, src_ref, ef_ref, emb_ref, wur_ref, wc_ref, bu_ref, br_ref,
             out_ref):
    src = src_ref[...]
    ef = ef_ref[...]
    et = et_ref[...]
    oh = (et == lax.broadcasted_iota(jnp.int32, (_B, _RP), 1)).astype(jnp.float32)
    g = jnp.dot(oh, emb_ref[...], preferred_element_type=jnp.float32)
    pur = jnp.dot(src, wur_ref[...], preferred_element_type=jnp.float32)
    u = jax.nn.sigmoid(g[:, :_D] * ef + pur[:, :_D] + bu_ref[...])
    r = jax.nn.sigmoid(g[:, _D:2 * _D] * ef + pur[:, _D:] + br_ref[...])
    c = jnp.tanh(
        g[:, 2 * _D:] * ef
        + jnp.dot(r * src, wc_ref[...], preferred_element_type=jnp.float32)
    )
    out_ref[...] = u * c + (1.0 - u) * src


def _tc_compute(et2, src, edge_feat, emb_all, wur, wc, bu, br):
    grid = _E // _B
    return pl.pallas_call(
        _tc_body,
        grid=(grid,),
        in_specs=[
            pl.BlockSpec((_B, 1), lambda i: (i, 0)),
            pl.BlockSpec((_B, _D), lambda i: (i, 0)),
            pl.BlockSpec((_B, _D), lambda i: (i, 0)),
            pl.BlockSpec((_RP, 3 * _D), lambda i: (0, 0)),
            pl.BlockSpec((_D, 2 * _D), lambda i: (0, 0)),
            pl.BlockSpec((_D, _D), lambda i: (0, 0)),
            pl.BlockSpec((1, _D), lambda i: (0, 0)),
            pl.BlockSpec((1, _D), lambda i: (0, 0)),
        ],
        out_specs=pl.BlockSpec((_B, _D), lambda i: (i, 0)),
        out_shape=jax.ShapeDtypeStruct((_E, _D), jnp.float32),
    )(et2, src, edge_feat, emb_all, wur, wc, bu, br)


def kernel(node_feat, edge_feat, src_idx, edge_type, emb_update, emb_reset,
           emb_candidate, W_update, b_update, W_reset, b_reset, W_candidate):
    src_idx = src_idx.astype(jnp.int32)
    edge_type = edge_type.astype(jnp.int32)
    src = _sc_gather(node_feat, src_idx)
    emb_all = jnp.pad(
        jnp.concatenate([emb_update, emb_reset, emb_candidate], axis=1),
        ((0, _RP - _R), (0, 0)),
    )
    wur = jnp.concatenate([W_update.T, W_reset.T], axis=1)
    return _tc_compute(
        edge_type.reshape(_E, 1),
        src,
        edge_feat,
        emb_all,
        wur,
        W_candidate.T,
        b_update.reshape(1, _D),
        b_reset.reshape(1, _D),
    )
